# baseline (speedup 1.0000x reference)
import functools
import math

import jax
import jax.numpy as jnp
from jax import lax
from jax.experimental import pallas as pl
from jax.experimental.pallas import tpu as pltpu
from jax.experimental.pallas import tpu_sc as plsc

B0, B1 = 4096, 200
D = 64
NW = 32
BW = B0 // NW
NBUF = 2
SCALE = math.sqrt(D)

_mesh = plsc.VectorSubcoreMesh(core_axis_name="c", subcore_axis_name="s")


@functools.partial(
    pl.kernel,
    mesh=_mesh,
    compiler_params=pltpu.CompilerParams(
        use_tc_tiling_on_sc=False, needs_layout_passes=False
    ),
    out_type=jax.ShapeDtypeStruct((B1, D // 8, NW, 8, BW), jnp.float32),
    scratch_types=[
        pltpu.VMEM((B1, BW), jnp.int32),
    ]
    + [pltpu.VMEM((BW,), jnp.int32) for _ in range(NBUF)]
    + [pltpu.VMEM((BW, 2 * D), jnp.float32) for _ in range(NBUF)]
    + [pltpu.VMEM((D // 8, 8, BW), jnp.float32) for _ in range(NBUF)]
    + [pltpu.SemaphoreType.DMA for _ in range(2 * NBUF)],
)
def _emb_lookup(xt_hbm, tab_hbm, out_hbm, idx_v, *rest):
    hbufs = rest[:NBUF]
    gbufs = rest[NBUF:2 * NBUF]
    tbufs = rest[2 * NBUF:3 * NBUF]
    gsem = rest[3 * NBUF:4 * NBUF]
    ssem = rest[4 * NBUF:]
    wid = lax.axis_index("s") * 2 + lax.axis_index("c")
    col0 = wid * BW
    pltpu.sync_copy(xt_hbm.at[:, pl.ds(col0, BW)], idx_v)

    lane = lax.iota(jnp.int32, 16)

    def fire_gather(s, i):
        for m in range(BW // 16):
            v = idx_v[s, pl.ds(16 * m, 16)]
            hbufs[i][pl.ds(16 * m, 16)] = lax.shift_right_logical(v, 1)
        pltpu.async_copy(tab_hbm.at[hbufs[i]], gbufs[i], gsem[i])

    def drain_gather(i):
        pltpu.make_async_copy(tab_hbm.at[hbufs[i]], gbufs[i], gsem[i]).wait()

    def transpose_scale(s, i):
        def tbody(m, _):
            v = idx_v[s, pl.ds(16 * m, 16)]
            colbase = jnp.bitwise_and(v, 1) * D
            rows = 16 * m + lane
            for f in range(D):
                g = plsc.load_gather(gbufs[i], [rows, colbase + f])
                tbufs[i][f // 8, f % 8, pl.ds(16 * m, 16)] = g * SCALE
            return ()

        lax.fori_loop(0, BW // 16, tbody, ())

    def fire_store(s, i):
        pltpu.async_copy(tbufs[i], out_hbm.at[s, :, wid], ssem[i])

    def drain_store(i):
        pltpu.make_async_copy(tbufs[i], out_hbm.at[0, :, wid], ssem[i]).wait()

    for i in range(NBUF):
        fire_gather(i, i)

    def body(t, _):
        s0 = t * NBUF
        for i in range(NBUF):
            drain_gather(i)
            transpose_scale(s0 + i, i)
            fire_store(s0 + i, i)
            if i >= 1:
                drain_store(i - 1)
                fire_gather(s0 + NBUF + (i - 1), i - 1)
        drain_store(NBUF - 1)
        fire_gather(s0 + NBUF + (NBUF - 1), NBUF - 1)
        return ()

    lax.fori_loop(0, B1 // NBUF - 1, body, ())

    s0 = B1 - NBUF
    for i in range(NBUF):
        drain_gather(i)
        transpose_scale(s0 + i, i)
        fire_store(s0 + i, i)
    for i in range(NBUF):
        drain_store(i)


def kernel(x, table):
    xt = x.astype(jnp.int32).T
    packed = table.reshape(table.shape[0] // 2, 2 * D)
    out5 = _emb_lookup(xt, packed)
    return out5.transpose(2, 4, 0, 1, 3).reshape(B0, B1, D)


# speedup vs baseline: 1.4616x; 1.4616x over previous
import functools
import math

import jax
import jax.numpy as jnp
from jax import lax
from jax.experimental import pallas as pl
from jax.experimental.pallas import tpu as pltpu
from jax.experimental.pallas import tpu_sc as plsc

B0, B1 = 4096, 200
D = 64
NW = 32
BW = B0 // NW
NBUF = 2
SCALE = math.sqrt(D)

_mesh = plsc.VectorSubcoreMesh(core_axis_name="c", subcore_axis_name="s")


@functools.partial(
    pl.kernel,
    mesh=_mesh,
    compiler_params=pltpu.CompilerParams(
        use_tc_tiling_on_sc=False, needs_layout_passes=False
    ),
    out_type=jax.ShapeDtypeStruct((B1, D // 8, NW, 8, BW), jnp.float32),
    scratch_types=[
        pltpu.VMEM((B1, BW), jnp.int32),
    ]
    + [pltpu.VMEM((BW,), jnp.int32) for _ in range(NBUF)]
    + [pltpu.VMEM((BW, 2 * D), jnp.float32) for _ in range(NBUF)]
    + [pltpu.VMEM((D // 8, 8, BW), jnp.float32) for _ in range(NBUF)]
    + [pltpu.VMEM((16,), jnp.float32)]
    + [pltpu.SemaphoreType.DMA for _ in range(2 * NBUF)],
)
def _emb_lookup(xt_hbm, tab_hbm, out_hbm, idx_v, *rest):
    hbufs = rest[:NBUF]
    gbufs = rest[NBUF:2 * NBUF]
    tbufs = rest[2 * NBUF:3 * NBUF]
    dummy = rest[3 * NBUF]
    gsem = rest[3 * NBUF + 1:3 * NBUF + 1 + NBUF]
    ssem = rest[3 * NBUF + 1 + NBUF:]
    wid = lax.axis_index("s") * 2 + lax.axis_index("c")
    col0 = wid * BW
    pltpu.sync_copy(xt_hbm.at[:, pl.ds(col0, BW)], idx_v)

    lane = lax.iota(jnp.int32, 16)

    def fire_gather(s, i):
        for m in range(BW // 16):
            v = idx_v[s, pl.ds(16 * m, 16)]
            hbufs[i][pl.ds(16 * m, 16)] = lax.shift_right_logical(v, 1)
        pltpu.async_copy(tab_hbm.at[hbufs[i]], gbufs[i], gsem[i])

    def drain_gather(i):
        pltpu.make_async_copy(tab_hbm.at[hbufs[i]], gbufs[i], gsem[i]).wait()

    def transpose_scale(s, i):
        def tbody(m, _):
            v = idx_v[s, pl.ds(16 * m, 16)]
            colbase = jnp.bitwise_and(v, 1) * D
            rows = 16 * m + lane

            @plsc.parallel_loop(0, D, unroll=8, carry=jnp.zeros((16,), jnp.float32))
            def fbody(f, c):
                del c
                g = plsc.load_gather(gbufs[i], [rows, colbase + f])
                gs = g * SCALE
                tbufs[i][f // 8, f % 8, pl.ds(16 * m, 16)] = gs
                return gs

            dummy[...] = fbody
            return ()

        lax.fori_loop(0, BW // 16, tbody, ())

    def fire_store(s, i):
        pltpu.async_copy(tbufs[i], out_hbm.at[s, :, wid], ssem[i])

    def drain_store(i):
        pltpu.make_async_copy(tbufs[i], out_hbm.at[0, :, wid], ssem[i]).wait()

    for i in range(NBUF):
        fire_gather(i, i)

    def body(t, _):
        s0 = t * NBUF
        for i in range(NBUF):
            drain_gather(i)
            transpose_scale(s0 + i, i)
            fire_store(s0 + i, i)
            if i >= 1:
                drain_store(i - 1)
                fire_gather(s0 + NBUF + (i - 1), i - 1)
        drain_store(NBUF - 1)
        fire_gather(s0 + NBUF + (NBUF - 1), NBUF - 1)
        return ()

    lax.fori_loop(0, B1 // NBUF - 1, body, ())

    s0 = B1 - NBUF
    for i in range(NBUF):
        drain_gather(i)
        transpose_scale(s0 + i, i)
        fire_store(s0 + i, i)
    for i in range(NBUF):
        drain_store(i)


def kernel(x, table):
    xt = x.astype(jnp.int32).T
    packed = table.reshape(table.shape[0] // 2, 2 * D)
    out5 = _emb_lookup(xt, packed)
    return out5.transpose(2, 4, 0, 1, 3).reshape(B0, B1, D)


# speedup vs baseline: 1.7180x; 1.1754x over previous
import functools
import math

import jax
import jax.numpy as jnp
from jax import lax
from jax.experimental import pallas as pl
from jax.experimental.pallas import tpu as pltpu
from jax.experimental.pallas import tpu_sc as plsc

B0, B1 = 4096, 200
D = 64
NW = 32
BW = B0 // NW
NBUF = 4
SCALE = math.sqrt(D)

_mesh = plsc.VectorSubcoreMesh(core_axis_name="c", subcore_axis_name="s")


@functools.partial(
    pl.kernel,
    mesh=_mesh,
    compiler_params=pltpu.CompilerParams(
        use_tc_tiling_on_sc=False, needs_layout_passes=False
    ),
    out_type=jax.ShapeDtypeStruct((B1, D // 8, NW, 8, BW), jnp.float32),
    scratch_types=[
        pltpu.VMEM((B1, BW), jnp.int32),
    ]
    + [pltpu.VMEM((BW, 2 * D), jnp.float32) for _ in range(NBUF)]
    + [pltpu.VMEM((D // 8, 8, BW), jnp.float32) for _ in range(NBUF)]
    + [pltpu.VMEM((16,), jnp.float32)]
    + [pltpu.SemaphoreType.DMA for _ in range(2 * NBUF)],
)
def _emb_lookup(xt_hbm, tab_hbm, out_hbm, idx_v, *rest):
    gbufs = rest[:NBUF]
    tbufs = rest[NBUF:2 * NBUF]
    dummy = rest[2 * NBUF]
    gsem = rest[2 * NBUF + 1:2 * NBUF + 1 + NBUF]
    ssem = rest[2 * NBUF + 1 + NBUF:]
    wid = lax.axis_index("s") * 2 + lax.axis_index("c")
    col0 = wid * BW
    pltpu.sync_copy(xt_hbm.at[:, pl.ds(col0, BW)], idx_v)

    lane = lax.iota(jnp.int32, 16)
    zero16 = jnp.zeros((16,), jnp.int32)

    def fire_gather(s, i):
        pltpu.async_copy(tab_hbm.at[idx_v.at[s]], gbufs[i], gsem[i])

    def drain_gather(i):
        pltpu.make_async_copy(tab_hbm.at[idx_v.at[0]], gbufs[i], gsem[i]).wait()

    def transpose_scale(i):
        def tbody(m, _):
            rows = 16 * m + lane

            @plsc.parallel_loop(0, D, unroll=8, carry=jnp.zeros((16,), jnp.float32))
            def fbody(f, c):
                del c
                g = plsc.load_gather(gbufs[i], [rows, zero16 + f])
                gs = g * SCALE
                tbufs[i][f // 8, f % 8, pl.ds(16 * m, 16)] = gs
                return gs

            dummy[...] = fbody
            return ()

        lax.fori_loop(0, BW // 16, tbody, ())

    def fire_store(s, i):
        pltpu.async_copy(tbufs[i], out_hbm.at[s, :, wid], ssem[i])

    def drain_store(i):
        pltpu.make_async_copy(tbufs[i], out_hbm.at[0, :, wid], ssem[i]).wait()

    for i in range(NBUF):
        fire_gather(i, i)

    def body(t, _):
        s0 = t * NBUF
        for i in range(NBUF):
            drain_gather(i)
            transpose_scale(i)
            fire_store(s0 + i, i)
            if i >= 1:
                drain_store(i - 1)
                fire_gather(s0 + NBUF + (i - 1), i - 1)
        drain_store(NBUF - 1)
        fire_gather(s0 + NBUF + (NBUF - 1), NBUF - 1)
        return ()

    lax.fori_loop(0, B1 // NBUF - 1, body, ())

    s0 = B1 - NBUF
    for i in range(NBUF):
        drain_gather(i)
        transpose_scale(i)
        fire_store(s0 + i, i)
    for i in range(NBUF):
        drain_store(i)


def kernel(x, table):
    xt = x.astype(jnp.int32).T
    wide = jnp.pad(table, ((0, 0), (0, 2 * D - table.shape[1])))
    out5 = _emb_lookup(xt, wide)
    return out5.transpose(2, 4, 0, 1, 3).reshape(B0, B1, D)


# speedup vs baseline: 2.6906x; 1.5661x over previous
import functools
import math

import jax
import jax.numpy as jnp
from jax import lax
from jax.experimental import pallas as pl
from jax.experimental.pallas import tpu as pltpu
from jax.experimental.pallas import tpu_sc as plsc

B0, B1 = 4096, 200
D = 64
NW = 32
BW = B0 // NW
NBUF = 4
SCALE = math.sqrt(D)

_mesh = plsc.VectorSubcoreMesh(core_axis_name="c", subcore_axis_name="s")


@functools.partial(
    pl.kernel,
    mesh=_mesh,
    compiler_params=pltpu.CompilerParams(
        use_tc_tiling_on_sc=False, needs_layout_passes=False
    ),
    out_type=jax.ShapeDtypeStruct((B1, D // 8, NW, 8, BW), jnp.float32),
    scratch_types=[
        pltpu.VMEM((B1, BW), jnp.int32),
    ]
    + [pltpu.VMEM((BW, 2 * D), jnp.float32) for _ in range(NBUF)]
    + [pltpu.VMEM((D // 8, 8, BW), jnp.float32) for _ in range(NBUF)]
    + [pltpu.VMEM((16,), jnp.float32)]
    + [pltpu.SemaphoreType.DMA for _ in range(2 * NBUF)],
)
def _emb_lookup(xt_hbm, tab_hbm, out_hbm, idx_v, *rest):
    gbufs = rest[:NBUF]
    tbufs = rest[NBUF:2 * NBUF]
    dummy = rest[2 * NBUF]
    gsem = rest[2 * NBUF + 1:2 * NBUF + 1 + NBUF]
    ssem = rest[2 * NBUF + 1 + NBUF:]
    wid = lax.axis_index("s") * 2 + lax.axis_index("c")
    col0 = wid * BW
    pltpu.sync_copy(xt_hbm.at[:, pl.ds(col0, BW)], idx_v)

    lane = lax.iota(jnp.int32, 16)
    zero16 = jnp.zeros((16,), jnp.int32)

    def fire_gather(s, i):
        pltpu.async_copy(tab_hbm.at[idx_v.at[s]], gbufs[i], gsem[i])

    def drain_gather(i):
        pltpu.make_async_copy(tab_hbm.at[idx_v.at[0]], gbufs[i], gsem[i]).wait()

    def transpose_scale(i):
        def tbody(m, _):
            rows = 16 * m + lane

            for fblock in range(D // 16):
                fbase = 16 * fblock

                @plsc.parallel_loop(
                    0, 16, unroll=8, carry=jnp.zeros((16,), jnp.float32)
                )
                def dbody(d, c):
                    del c
                    fb = fbase + jnp.bitwise_and(d + lane, 15)
                    g = plsc.load_gather(gbufs[i], [rows, fb])
                    gs = g * SCALE
                    plsc.store_scatter(
                        tbufs[i],
                        [lax.shift_right_logical(fb, 3),
                         jnp.bitwise_and(fb, 7),
                         rows],
                        gs,
                    )
                    return gs

                dummy[...] = dbody
            return ()

        lax.fori_loop(0, BW // 16, tbody, ())

    def fire_store(s, i):
        pltpu.async_copy(tbufs[i], out_hbm.at[s, :, wid], ssem[i])

    def drain_store(i):
        pltpu.make_async_copy(tbufs[i], out_hbm.at[0, :, wid], ssem[i]).wait()

    for i in range(NBUF):
        fire_gather(i, i)

    def body(t, _):
        s0 = t * NBUF
        for i in range(NBUF):
            drain_gather(i)
            transpose_scale(i)
            fire_store(s0 + i, i)
            if i >= 1:
                drain_store(i - 1)
                fire_gather(s0 + NBUF + (i - 1), i - 1)
        drain_store(NBUF - 1)
        fire_gather(s0 + NBUF + (NBUF - 1), NBUF - 1)
        return ()

    lax.fori_loop(0, B1 // NBUF - 1, body, ())

    s0 = B1 - NBUF
    for i in range(NBUF):
        drain_gather(i)
        transpose_scale(i)
        fire_store(s0 + i, i)
    for i in range(NBUF):
        drain_store(i)


def kernel(x, table):
    xt = x.astype(jnp.int32).T
    wide = jnp.pad(table, ((0, 0), (0, 2 * D - table.shape[1])))
    out5 = _emb_lookup(xt, wide)
    return out5.transpose(2, 4, 0, 1, 3).reshape(B0, B1, D)
